# baseline (speedup 1.0000x reference)
import jax
import jax.numpy as jnp
from jax.experimental import pallas as pl
from jax.experimental.pallas import tpu as pltpu

_TN = 16384
_CHUNK = 128
_IDB = 13


def _round_up(x, m):
    return ((x + m - 1) // m) * m


def _gather_dot_kernel(w_ref, pids_ref, uids_ref, ptab_ref, utab_ref,
                       out_ref, c_ref):
    tn = out_ref.shape[1]

    def chunk_body(c, carry):
        base = c * _CHUNK
        hbase = c * (_CHUNK // 2)
        for j in range(_CHUNK // 8):
            rows = []
            for i in range(4):
                w = w_ref[0, hbase + j * 4 + i]
                p = w & ((1 << _IDB) - 1)
                u = jax.lax.shift_right_logical(w, _IDB)
                rows.append(ptab_ref[p] * utab_ref[u])
            for i in range(4):
                p = pids_ref[0, hbase + j * 4 + i]
                u = uids_ref[0, hbase + j * 4 + i]
                rows.append(ptab_ref[p] * utab_ref[u])
            blk = jnp.concatenate(rows, axis=0)
            c_ref[pl.ds(pl.multiple_of(base + j * 8, 8), 8), :] = blk
        return carry

    half = tn // (2 * _CHUNK)
    ones = jnp.ones((1, 128), jnp.float32)
    dims = (((1,), (1,)), ((), ()))

    jax.lax.fori_loop(0, half, chunk_body, 0)
    out_ref[0, pl.ds(0, tn // 2)] = jax.lax.dot_general(
        ones, c_ref[pl.ds(0, tn // 2), :], dims,
        preferred_element_type=jnp.float32)[0]
    jax.lax.fori_loop(half, 2 * half, chunk_body, 0)
    out_ref[0, pl.ds(tn // 2, tn // 2)] = jax.lax.dot_general(
        ones, c_ref[pl.ds(tn // 2, tn // 2), :], dims,
        preferred_element_type=jnp.float32)[0]


def kernel(products, users, product_embedding, user_embedding):
    n = products.shape[0]
    p_rows, d = product_embedding.shape
    u_rows, d_u = user_embedding.shape
    assert d == d_u == 128

    n_pad = _round_up(n, _TN)
    num_tiles = n_pad // _TN

    def clamp_pad(ids, rows):
        ids = jnp.clip(jnp.asarray(ids).astype(jnp.int32), 0, rows - 1)
        return jnp.pad(ids, (0, n_pad - n)).reshape(num_tiles, _TN // 8, 8)

    p2d = clamp_pad(products, p_rows)
    u2d = clamp_pad(users, u_rows)
    packed = (p2d[..., :4] | (u2d[..., :4] << _IDB)).reshape(
        num_tiles, 1, _TN // 2)
    prod_ids = p2d[..., 4:].reshape(num_tiles, 1, _TN // 2)
    user_ids = u2d[..., 4:].reshape(num_tiles, 1, _TN // 2)

    ptab = product_embedding.astype(jnp.float32).reshape(p_rows, 1, d)
    utab = user_embedding.astype(jnp.float32).reshape(u_rows, 1, d)

    table_bytes = (p_rows + u_rows) * d * 4
    vmem_limit = min(int(2 * table_bytes + 4 * _TN * 128 * 4 + (8 << 20)),
                     60 << 20)

    cost = pl.CostEstimate(
        flops=2 * n_pad * d + 2 * n_pad * d,
        transcendentals=0,
        bytes_accessed=2 * n_pad * 4 + 2 * table_bytes + n_pad * 4,
    )

    out = pl.pallas_call(
        _gather_dot_kernel,
        out_shape=jax.ShapeDtypeStruct((num_tiles, 1, _TN), jnp.float32),
        grid=(num_tiles,),
        in_specs=[
            pl.BlockSpec((None, 1, _TN // 2), lambda t: (t, 0, 0),
                         memory_space=pltpu.SMEM),
            pl.BlockSpec((None, 1, _TN // 2), lambda t: (t, 0, 0),
                         memory_space=pltpu.SMEM),
            pl.BlockSpec((None, 1, _TN // 2), lambda t: (t, 0, 0),
                         memory_space=pltpu.SMEM),
            pl.BlockSpec((p_rows, 1, d), lambda t: (0, 0, 0)),
            pl.BlockSpec((u_rows, 1, d), lambda t: (0, 0, 0)),
        ],
        out_specs=pl.BlockSpec((None, 1, _TN), lambda t: (t, 0, 0)),
        scratch_shapes=[pltpu.VMEM((_TN, d), jnp.float32)],
        compiler_params=pltpu.CompilerParams(
            dimension_semantics=("parallel",),
            vmem_limit_bytes=vmem_limit,
        ),
        cost_estimate=cost,
    )(packed, prod_ids, user_ids, ptab, utab)
    return out.reshape(n_pad)[:n]


# speedup vs baseline: 1.3449x; 1.3449x over previous
import jax
import jax.numpy as jnp
from jax.experimental import pallas as pl
from jax.experimental.pallas import tpu as pltpu

_TN = 16384
_CHUNK = 256


def _round_up(x, m):
    return ((x + m - 1) // m) * m


def _gather_dot_kernel(pids_ref, uids_ref, ptab_ref, utab_ref, out_ref, c_ref):
    tn = out_ref.shape[1]

    def chunk_body(c, carry):
        base = c * _CHUNK
        for j in range(_CHUNK // 8):
            rows = []
            for i in range(8):
                p = pids_ref[0, base + j * 8 + i]
                u = uids_ref[0, base + j * 8 + i]
                rows.append(ptab_ref[p] * utab_ref[u])
            blk = jnp.concatenate(rows, axis=0)
            c_ref[pl.ds(pl.multiple_of(base + j * 8, 8), 8), :] = blk
        return carry

    half = tn // (2 * _CHUNK)
    ones = jnp.ones((1, 128), jnp.float32)
    dims = (((1,), (1,)), ((), ()))

    jax.lax.fori_loop(0, half, chunk_body, 0)
    out_ref[0, pl.ds(0, tn // 2)] = jax.lax.dot_general(
        ones, c_ref[pl.ds(0, tn // 2), :], dims,
        preferred_element_type=jnp.float32)[0]
    jax.lax.fori_loop(half, 2 * half, chunk_body, 0)
    out_ref[0, pl.ds(tn // 2, tn // 2)] = jax.lax.dot_general(
        ones, c_ref[pl.ds(tn // 2, tn // 2), :], dims,
        preferred_element_type=jnp.float32)[0]


def kernel(products, users, product_embedding, user_embedding):
    n = products.shape[0]
    p_rows, d = product_embedding.shape
    u_rows, d_u = user_embedding.shape
    assert d == d_u == 128

    n_pad = _round_up(n, _TN)
    num_tiles = n_pad // _TN

    def prep_ids(ids, rows):
        ids = jnp.clip(jnp.asarray(ids).astype(jnp.int32), 0, rows - 1)
        ids = jnp.pad(ids, (0, n_pad - n))
        return ids.reshape(num_tiles, 1, _TN)

    prod_ids = prep_ids(products, p_rows)
    user_ids = prep_ids(users, u_rows)

    ptab = product_embedding.astype(jnp.float32).reshape(p_rows, 1, d)
    utab = user_embedding.astype(jnp.float32).reshape(u_rows, 1, d)

    table_bytes = (p_rows + u_rows) * d * 4
    vmem_limit = min(int(2 * table_bytes + 4 * _TN * 128 * 4 + (8 << 20)),
                     60 << 20)

    cost = pl.CostEstimate(
        flops=2 * n_pad * d + 2 * n_pad * d,
        transcendentals=0,
        bytes_accessed=2 * n_pad * 4 + 2 * table_bytes + n_pad * 4,
    )

    out = pl.pallas_call(
        _gather_dot_kernel,
        out_shape=jax.ShapeDtypeStruct((num_tiles, 1, _TN), jnp.float32),
        grid=(num_tiles,),
        in_specs=[
            pl.BlockSpec((None, 1, _TN), lambda t: (t, 0, 0),
                         memory_space=pltpu.SMEM),
            pl.BlockSpec((None, 1, _TN), lambda t: (t, 0, 0),
                         memory_space=pltpu.SMEM),
            pl.BlockSpec((p_rows, 1, d), lambda t: (0, 0, 0)),
            pl.BlockSpec((u_rows, 1, d), lambda t: (0, 0, 0)),
        ],
        out_specs=pl.BlockSpec((None, 1, _TN), lambda t: (t, 0, 0)),
        scratch_shapes=[pltpu.VMEM((_TN, d), jnp.float32)],
        compiler_params=pltpu.CompilerParams(
            dimension_semantics=("parallel",),
            vmem_limit_bytes=vmem_limit,
        ),
        cost_estimate=cost,
    )(prod_ids, user_ids, ptab, utab)
    return out.reshape(n_pad)[:n]


# speedup vs baseline: 1.3578x; 1.0096x over previous
import jax
import jax.numpy as jnp
from jax.experimental import pallas as pl
from jax.experimental.pallas import tpu as pltpu

_TN = 16384
_CHUNK = 512


def _round_up(x, m):
    return ((x + m - 1) // m) * m


def _gather_dot_kernel(pids_ref, uids_ref, ptab_ref, utab_ref, out_ref, c_ref):
    tn = out_ref.shape[1]

    def chunk_body(c, carry):
        base = c * _CHUNK
        for j in range(_CHUNK // 8):
            rows = []
            for i in range(8):
                p = pids_ref[0, base + j * 8 + i]
                u = uids_ref[0, base + j * 8 + i]
                rows.append(ptab_ref[p] * utab_ref[u])
            blk = jnp.concatenate(rows, axis=0)
            c_ref[pl.ds(pl.multiple_of(base + j * 8, 8), 8), :] = blk
        return carry

    half = tn // (2 * _CHUNK)
    ones = jnp.ones((1, 128), jnp.float32)
    dims = (((1,), (1,)), ((), ()))

    jax.lax.fori_loop(0, half, chunk_body, 0)
    out_ref[0, pl.ds(0, tn // 2)] = jax.lax.dot_general(
        ones, c_ref[pl.ds(0, tn // 2), :], dims,
        preferred_element_type=jnp.float32)[0]
    jax.lax.fori_loop(half, 2 * half, chunk_body, 0)
    out_ref[0, pl.ds(tn // 2, tn // 2)] = jax.lax.dot_general(
        ones, c_ref[pl.ds(tn // 2, tn // 2), :], dims,
        preferred_element_type=jnp.float32)[0]


def kernel(products, users, product_embedding, user_embedding):
    n = products.shape[0]
    p_rows, d = product_embedding.shape
    u_rows, d_u = user_embedding.shape
    assert d == d_u == 128

    n_pad = _round_up(n, _TN)
    num_tiles = n_pad // _TN

    def prep_ids(ids, rows):
        ids = jnp.clip(jnp.asarray(ids).astype(jnp.int32), 0, rows - 1)
        ids = jnp.pad(ids, (0, n_pad - n))
        return ids.reshape(num_tiles, 1, _TN)

    prod_ids = prep_ids(products, p_rows)
    user_ids = prep_ids(users, u_rows)

    ptab = product_embedding.astype(jnp.float32).reshape(p_rows, 1, d)
    utab = user_embedding.astype(jnp.float32).reshape(u_rows, 1, d)

    table_bytes = (p_rows + u_rows) * d * 4
    vmem_limit = min(int(2 * table_bytes + 4 * _TN * 128 * 4 + (8 << 20)),
                     60 << 20)

    cost = pl.CostEstimate(
        flops=2 * n_pad * d + 2 * n_pad * d,
        transcendentals=0,
        bytes_accessed=2 * n_pad * 4 + 2 * table_bytes + n_pad * 4,
    )

    out = pl.pallas_call(
        _gather_dot_kernel,
        out_shape=jax.ShapeDtypeStruct((num_tiles, 1, _TN), jnp.float32),
        grid=(num_tiles,),
        in_specs=[
            pl.BlockSpec((None, 1, _TN), lambda t: (t, 0, 0),
                         memory_space=pltpu.SMEM),
            pl.BlockSpec((None, 1, _TN), lambda t: (t, 0, 0),
                         memory_space=pltpu.SMEM),
            pl.BlockSpec((p_rows, 1, d), lambda t: (0, 0, 0)),
            pl.BlockSpec((u_rows, 1, d), lambda t: (0, 0, 0)),
        ],
        out_specs=pl.BlockSpec((None, 1, _TN), lambda t: (t, 0, 0)),
        scratch_shapes=[pltpu.VMEM((_TN, d), jnp.float32)],
        compiler_params=pltpu.CompilerParams(
            dimension_semantics=("parallel",),
            vmem_limit_bytes=vmem_limit,
        ),
        cost_estimate=cost,
    )(prod_ids, user_ids, ptab, utab)
    return out.reshape(n_pad)[:n]


# speedup vs baseline: 1.3651x; 1.0054x over previous
import jax
import jax.numpy as jnp
from jax.experimental import pallas as pl
from jax.experimental.pallas import tpu as pltpu

_TN = 16384
_CHUNK = 1024


def _round_up(x, m):
    return ((x + m - 1) // m) * m


def _gather_dot_kernel(pids_ref, uids_ref, ptab_ref, utab_ref, out_ref, c_ref):
    tn = out_ref.shape[1]

    def chunk_body(c, carry):
        base = c * _CHUNK
        for j in range(_CHUNK // 8):
            rows = []
            for i in range(8):
                p = pids_ref[0, base + j * 8 + i]
                u = uids_ref[0, base + j * 8 + i]
                rows.append(ptab_ref[p] * utab_ref[u])
            blk = jnp.concatenate(rows, axis=0)
            c_ref[pl.ds(pl.multiple_of(base + j * 8, 8), 8), :] = blk
        return carry

    half = tn // (2 * _CHUNK)
    ones = jnp.ones((1, 128), jnp.float32)
    dims = (((1,), (1,)), ((), ()))

    jax.lax.fori_loop(0, half, chunk_body, 0)
    out_ref[0, pl.ds(0, tn // 2)] = jax.lax.dot_general(
        ones, c_ref[pl.ds(0, tn // 2), :], dims,
        preferred_element_type=jnp.float32)[0]
    jax.lax.fori_loop(half, 2 * half, chunk_body, 0)
    out_ref[0, pl.ds(tn // 2, tn // 2)] = jax.lax.dot_general(
        ones, c_ref[pl.ds(tn // 2, tn // 2), :], dims,
        preferred_element_type=jnp.float32)[0]


def kernel(products, users, product_embedding, user_embedding):
    n = products.shape[0]
    p_rows, d = product_embedding.shape
    u_rows, d_u = user_embedding.shape
    assert d == d_u == 128

    n_pad = _round_up(n, _TN)
    num_tiles = n_pad // _TN

    def prep_ids(ids, rows):
        ids = jnp.clip(jnp.asarray(ids).astype(jnp.int32), 0, rows - 1)
        ids = jnp.pad(ids, (0, n_pad - n))
        return ids.reshape(num_tiles, 1, _TN)

    prod_ids = prep_ids(products, p_rows)
    user_ids = prep_ids(users, u_rows)

    ptab = product_embedding.astype(jnp.float32).reshape(p_rows, 1, d)
    utab = user_embedding.astype(jnp.float32).reshape(u_rows, 1, d)

    table_bytes = (p_rows + u_rows) * d * 4
    vmem_limit = min(int(2 * table_bytes + 4 * _TN * 128 * 4 + (8 << 20)),
                     60 << 20)

    cost = pl.CostEstimate(
        flops=2 * n_pad * d + 2 * n_pad * d,
        transcendentals=0,
        bytes_accessed=2 * n_pad * 4 + 2 * table_bytes + n_pad * 4,
    )

    out = pl.pallas_call(
        _gather_dot_kernel,
        out_shape=jax.ShapeDtypeStruct((num_tiles, 1, _TN), jnp.float32),
        grid=(num_tiles,),
        in_specs=[
            pl.BlockSpec((None, 1, _TN), lambda t: (t, 0, 0),
                         memory_space=pltpu.SMEM),
            pl.BlockSpec((None, 1, _TN), lambda t: (t, 0, 0),
                         memory_space=pltpu.SMEM),
            pl.BlockSpec((p_rows, 1, d), lambda t: (0, 0, 0)),
            pl.BlockSpec((u_rows, 1, d), lambda t: (0, 0, 0)),
        ],
        out_specs=pl.BlockSpec((None, 1, _TN), lambda t: (t, 0, 0)),
        scratch_shapes=[pltpu.VMEM((_TN, d), jnp.float32)],
        compiler_params=pltpu.CompilerParams(
            dimension_semantics=("parallel",),
            vmem_limit_bytes=vmem_limit,
        ),
        cost_estimate=cost,
    )(prod_ids, user_ids, ptab, utab)
    return out.reshape(n_pad)[:n]


# speedup vs baseline: 1.3724x; 1.0053x over previous
import jax
import jax.numpy as jnp
from jax.experimental import pallas as pl
from jax.experimental.pallas import tpu as pltpu

_TN = 32768
_CHUNK = 1024


def _round_up(x, m):
    return ((x + m - 1) // m) * m


def _gather_dot_kernel(pids_ref, uids_ref, ptab_ref, utab_ref, out_ref, c_ref):
    tn = out_ref.shape[1]

    def chunk_body(c, carry):
        base = c * _CHUNK
        for j in range(_CHUNK // 8):
            rows = []
            for i in range(8):
                p = pids_ref[0, base + j * 8 + i]
                u = uids_ref[0, base + j * 8 + i]
                rows.append(ptab_ref[p] * utab_ref[u])
            blk = jnp.concatenate(rows, axis=0)
            c_ref[pl.ds(pl.multiple_of(base + j * 8, 8), 8), :] = blk
        return carry

    half = tn // (2 * _CHUNK)
    ones = jnp.ones((1, 128), jnp.float32)
    dims = (((1,), (1,)), ((), ()))

    jax.lax.fori_loop(0, half, chunk_body, 0)
    out_ref[0, pl.ds(0, tn // 2)] = jax.lax.dot_general(
        ones, c_ref[pl.ds(0, tn // 2), :], dims,
        preferred_element_type=jnp.float32)[0]
    jax.lax.fori_loop(half, 2 * half, chunk_body, 0)
    out_ref[0, pl.ds(tn // 2, tn // 2)] = jax.lax.dot_general(
        ones, c_ref[pl.ds(tn // 2, tn // 2), :], dims,
        preferred_element_type=jnp.float32)[0]


def kernel(products, users, product_embedding, user_embedding):
    n = products.shape[0]
    p_rows, d = product_embedding.shape
    u_rows, d_u = user_embedding.shape
    assert d == d_u == 128

    n_pad = _round_up(n, _TN)
    num_tiles = n_pad // _TN

    def prep_ids(ids, rows):
        ids = jnp.clip(jnp.asarray(ids).astype(jnp.int32), 0, rows - 1)
        ids = jnp.pad(ids, (0, n_pad - n))
        return ids.reshape(num_tiles, 1, _TN)

    prod_ids = prep_ids(products, p_rows)
    user_ids = prep_ids(users, u_rows)

    ptab = product_embedding.astype(jnp.float32).reshape(p_rows, 1, d)
    utab = user_embedding.astype(jnp.float32).reshape(u_rows, 1, d)

    table_bytes = (p_rows + u_rows) * d * 4
    vmem_limit = min(int(2 * table_bytes + 4 * _TN * 128 * 4 + (8 << 20)),
                     60 << 20)

    cost = pl.CostEstimate(
        flops=2 * n_pad * d + 2 * n_pad * d,
        transcendentals=0,
        bytes_accessed=2 * n_pad * 4 + 2 * table_bytes + n_pad * 4,
    )

    out = pl.pallas_call(
        _gather_dot_kernel,
        out_shape=jax.ShapeDtypeStruct((num_tiles, 1, _TN), jnp.float32),
        grid=(num_tiles,),
        in_specs=[
            pl.BlockSpec((None, 1, _TN), lambda t: (t, 0, 0),
                         memory_space=pltpu.SMEM),
            pl.BlockSpec((None, 1, _TN), lambda t: (t, 0, 0),
                         memory_space=pltpu.SMEM),
            pl.BlockSpec((p_rows, 1, d), lambda t: (0, 0, 0)),
            pl.BlockSpec((u_rows, 1, d), lambda t: (0, 0, 0)),
        ],
        out_specs=pl.BlockSpec((None, 1, _TN), lambda t: (t, 0, 0)),
        scratch_shapes=[pltpu.VMEM((_TN, d), jnp.float32)],
        compiler_params=pltpu.CompilerParams(
            dimension_semantics=("parallel",),
            vmem_limit_bytes=vmem_limit,
        ),
        cost_estimate=cost,
    )(prod_ids, user_ids, ptab, utab)
    return out.reshape(n_pad)[:n]


# speedup vs baseline: 1.3753x; 1.0021x over previous
import jax
import jax.numpy as jnp
from jax.experimental import pallas as pl
from jax.experimental.pallas import tpu as pltpu

_TN = 32768
_CHUNK = 2048


def _round_up(x, m):
    return ((x + m - 1) // m) * m


def _gather_dot_kernel(pids_ref, uids_ref, ptab_ref, utab_ref, out_ref, c_ref):
    tn = out_ref.shape[1]

    def chunk_body(c, carry):
        base = c * _CHUNK
        for j in range(_CHUNK // 8):
            rows = []
            for i in range(8):
                p = pids_ref[0, base + j * 8 + i]
                u = uids_ref[0, base + j * 8 + i]
                rows.append(ptab_ref[p] * utab_ref[u])
            blk = jnp.concatenate(rows, axis=0)
            c_ref[pl.ds(pl.multiple_of(base + j * 8, 8), 8), :] = blk
        return carry

    half = tn // (2 * _CHUNK)
    ones = jnp.ones((1, 128), jnp.float32)
    dims = (((1,), (1,)), ((), ()))

    jax.lax.fori_loop(0, half, chunk_body, 0)
    out_ref[0, pl.ds(0, tn // 2)] = jax.lax.dot_general(
        ones, c_ref[pl.ds(0, tn // 2), :], dims,
        preferred_element_type=jnp.float32)[0]
    jax.lax.fori_loop(half, 2 * half, chunk_body, 0)
    out_ref[0, pl.ds(tn // 2, tn // 2)] = jax.lax.dot_general(
        ones, c_ref[pl.ds(tn // 2, tn // 2), :], dims,
        preferred_element_type=jnp.float32)[0]


def kernel(products, users, product_embedding, user_embedding):
    n = products.shape[0]
    p_rows, d = product_embedding.shape
    u_rows, d_u = user_embedding.shape
    assert d == d_u == 128

    n_pad = _round_up(n, _TN)
    num_tiles = n_pad // _TN

    def prep_ids(ids, rows):
        ids = jnp.clip(jnp.asarray(ids).astype(jnp.int32), 0, rows - 1)
        ids = jnp.pad(ids, (0, n_pad - n))
        return ids.reshape(num_tiles, 1, _TN)

    prod_ids = prep_ids(products, p_rows)
    user_ids = prep_ids(users, u_rows)

    ptab = product_embedding.astype(jnp.float32).reshape(p_rows, 1, d)
    utab = user_embedding.astype(jnp.float32).reshape(u_rows, 1, d)

    table_bytes = (p_rows + u_rows) * d * 4
    vmem_limit = min(int(2 * table_bytes + 4 * _TN * 128 * 4 + (8 << 20)),
                     60 << 20)

    cost = pl.CostEstimate(
        flops=2 * n_pad * d + 2 * n_pad * d,
        transcendentals=0,
        bytes_accessed=2 * n_pad * 4 + 2 * table_bytes + n_pad * 4,
    )

    out = pl.pallas_call(
        _gather_dot_kernel,
        out_shape=jax.ShapeDtypeStruct((num_tiles, 1, _TN), jnp.float32),
        grid=(num_tiles,),
        in_specs=[
            pl.BlockSpec((None, 1, _TN), lambda t: (t, 0, 0),
                         memory_space=pltpu.SMEM),
            pl.BlockSpec((None, 1, _TN), lambda t: (t, 0, 0),
                         memory_space=pltpu.SMEM),
            pl.BlockSpec((p_rows, 1, d), lambda t: (0, 0, 0)),
            pl.BlockSpec((u_rows, 1, d), lambda t: (0, 0, 0)),
        ],
        out_specs=pl.BlockSpec((None, 1, _TN), lambda t: (t, 0, 0)),
        scratch_shapes=[pltpu.VMEM((_TN, d), jnp.float32)],
        compiler_params=pltpu.CompilerParams(
            dimension_semantics=("parallel",),
            vmem_limit_bytes=vmem_limit,
        ),
        cost_estimate=cost,
    )(prod_ids, user_ids, ptab, utab)
    return out.reshape(n_pad)[:n]


# speedup vs baseline: 1.3769x; 1.0012x over previous
import jax
import jax.numpy as jnp
from jax.experimental import pallas as pl
from jax.experimental.pallas import tpu as pltpu

_TN = 32768
_CHUNK = 4096


def _round_up(x, m):
    return ((x + m - 1) // m) * m


def _gather_dot_kernel(pids_ref, uids_ref, ptab_ref, utab_ref, out_ref, c_ref):
    tn = out_ref.shape[1]

    def chunk_body(c, carry):
        base = c * _CHUNK
        for j in range(_CHUNK // 8):
            rows = []
            for i in range(8):
                p = pids_ref[0, base + j * 8 + i]
                u = uids_ref[0, base + j * 8 + i]
                rows.append(ptab_ref[p] * utab_ref[u])
            blk = jnp.concatenate(rows, axis=0)
            c_ref[pl.ds(pl.multiple_of(base + j * 8, 8), 8), :] = blk
        return carry

    half = tn // (2 * _CHUNK)
    ones = jnp.ones((1, 128), jnp.float32)
    dims = (((1,), (1,)), ((), ()))

    jax.lax.fori_loop(0, half, chunk_body, 0)
    out_ref[0, pl.ds(0, tn // 2)] = jax.lax.dot_general(
        ones, c_ref[pl.ds(0, tn // 2), :], dims,
        preferred_element_type=jnp.float32)[0]
    jax.lax.fori_loop(half, 2 * half, chunk_body, 0)
    out_ref[0, pl.ds(tn // 2, tn // 2)] = jax.lax.dot_general(
        ones, c_ref[pl.ds(tn // 2, tn // 2), :], dims,
        preferred_element_type=jnp.float32)[0]


def kernel(products, users, product_embedding, user_embedding):
    n = products.shape[0]
    p_rows, d = product_embedding.shape
    u_rows, d_u = user_embedding.shape
    assert d == d_u == 128

    n_pad = _round_up(n, _TN)
    num_tiles = n_pad // _TN

    def prep_ids(ids, rows):
        ids = jnp.clip(jnp.asarray(ids).astype(jnp.int32), 0, rows - 1)
        ids = jnp.pad(ids, (0, n_pad - n))
        return ids.reshape(num_tiles, 1, _TN)

    prod_ids = prep_ids(products, p_rows)
    user_ids = prep_ids(users, u_rows)

    ptab = product_embedding.astype(jnp.float32).reshape(p_rows, 1, d)
    utab = user_embedding.astype(jnp.float32).reshape(u_rows, 1, d)

    table_bytes = (p_rows + u_rows) * d * 4
    vmem_limit = min(int(2 * table_bytes + 4 * _TN * 128 * 4 + (8 << 20)),
                     60 << 20)

    cost = pl.CostEstimate(
        flops=2 * n_pad * d + 2 * n_pad * d,
        transcendentals=0,
        bytes_accessed=2 * n_pad * 4 + 2 * table_bytes + n_pad * 4,
    )

    out = pl.pallas_call(
        _gather_dot_kernel,
        out_shape=jax.ShapeDtypeStruct((num_tiles, 1, _TN), jnp.float32),
        grid=(num_tiles,),
        in_specs=[
            pl.BlockSpec((None, 1, _TN), lambda t: (t, 0, 0),
                         memory_space=pltpu.SMEM),
            pl.BlockSpec((None, 1, _TN), lambda t: (t, 0, 0),
                         memory_space=pltpu.SMEM),
            pl.BlockSpec((p_rows, 1, d), lambda t: (0, 0, 0)),
            pl.BlockSpec((u_rows, 1, d), lambda t: (0, 0, 0)),
        ],
        out_specs=pl.BlockSpec((None, 1, _TN), lambda t: (t, 0, 0)),
        scratch_shapes=[pltpu.VMEM((_TN, d), jnp.float32)],
        compiler_params=pltpu.CompilerParams(
            dimension_semantics=("parallel",),
            vmem_limit_bytes=vmem_limit,
        ),
        cost_estimate=cost,
    )(prod_ids, user_ids, ptab, utab)
    return out.reshape(n_pad)[:n]
